# baseline (speedup 1.0000x reference)
import functools

import jax
import jax.numpy as jnp
from jax import lax
from jax.experimental import pallas as pl
from jax.experimental.pallas import tpu as pltpu
from jax.experimental.pallas import tpu_sc as plsc

N = 10000
E = 320000
D = 128
DH = D // 2
NC = 2
NS = 16
NW = NC * NS
G = 128
NGROUPS = 2560
EPAD = NGROUPS * G
GPC = NGROUPS // NS
GPT = NGROUPS // NW
NP = N + 8
ZMAIN = 624
W8 = 8
U = 5

_mesh = plsc.VectorSubcoreMesh(core_axis_name="c", subcore_axis_name="s")


@functools.partial(
    pl.kernel,
    out_type=[
        jax.ShapeDtypeStruct((NC * N, DH), jnp.float32),
        jax.ShapeDtypeStruct((NC * N, W8), jnp.float32),
    ],
    mesh=_mesh,
    compiler_params=pltpu.CompilerParams(use_tc_tiling_on_sc=False),
    scratch_types=[
        pltpu.VMEM((GPC, G), jnp.int32),
        pltpu.VMEM((GPC, G), jnp.int32),
        pltpu.VMEM((U, G, DH), jnp.float32),
        pltpu.VMEM((G, W8), jnp.float32),
        pltpu.VMEM_SHARED((NP, DH), jnp.float32),
        pltpu.VMEM_SHARED((NP, W8), jnp.float32),
        [pltpu.SemaphoreType.DMA] * U,
        pltpu.SemaphoreType.DMA,
        pltpu.SemaphoreType.DMA,
    ],
)
def _sc_pass1(xs_hbm, srcx_hbm, dst_hbm, z64_hbm, z8_hbm, ones_hbm,
              aggp_hbm, cntp_hbm,
              src_v, dst_v, rows_v, ones_v, acc_s, cnt_s, gsems, ssem, osem):
    cid = lax.axis_index("c")
    sid = lax.axis_index("s")

    rb = sid * ZMAIN
    pltpu.sync_copy(z64_hbm, acc_s.at[pl.ds(rb, ZMAIN)])
    pltpu.sync_copy(z8_hbm, cnt_s.at[pl.ds(rb, ZMAIN)])

    @pl.when(sid == NS - 1)
    def _():
        tb = NS * ZMAIN
        pltpu.sync_copy(z64_hbm.at[pl.ds(0, NP - tb)], acc_s.at[pl.ds(tb, NP - tb)])
        pltpu.sync_copy(z8_hbm.at[pl.ds(0, NP - tb)], cnt_s.at[pl.ds(tb, NP - tb)])

    gb = sid * GPC
    pltpu.sync_copy(srcx_hbm.at[pl.ds(cid * NGROUPS + gb, GPC)], src_v)
    pltpu.sync_copy(dst_hbm.at[pl.ds(gb, GPC)], dst_v)
    pltpu.sync_copy(ones_hbm, ones_v)

    plsc.subcore_barrier()

    NB = GPC // U

    def body(i, carry):
        b = i * U
        dg = [
            pltpu.async_copy(
                xs_hbm.at[src_v.at[b + k]], rows_v.at[k], gsems[k])
            for k in range(U)
        ]
        ds = []
        for k in range(U):
            dg[k].wait()
            ds.append(pltpu.async_copy(
                rows_v.at[k], acc_s.at[dst_v.at[b + k]], ssem, add=True))

        @pl.when((cid == 0) == (i < NB // 2))
        def _():
            do = [
                pltpu.async_copy(
                    ones_v, cnt_s.at[dst_v.at[b + k]], osem, add=True)
                for k in range(U)
            ]
            for d in do:
                d.wait()

        for d in ds:
            d.wait()
        return carry

    lax.fori_loop(0, NB, body, 0)
    plsc.subcore_barrier()

    pltpu.sync_copy(acc_s.at[pl.ds(rb, ZMAIN)], aggp_hbm.at[pl.ds(cid * N + rb, ZMAIN)])
    pltpu.sync_copy(cnt_s.at[pl.ds(rb, ZMAIN)], cntp_hbm.at[pl.ds(cid * N + rb, ZMAIN)])

    @pl.when(sid == NS - 1)
    def _():
        tb = NS * ZMAIN
        pltpu.sync_copy(acc_s.at[pl.ds(tb, N - tb)], aggp_hbm.at[pl.ds(cid * N + tb, N - tb)])
        pltpu.sync_copy(cnt_s.at[pl.ds(tb, N - tb)], cntp_hbm.at[pl.ds(cid * N + tb, N - tb)])


@functools.partial(
    pl.kernel,
    out_type=[jax.ShapeDtypeStruct((NC * N, W8), jnp.float32)],
    mesh=_mesh,
    compiler_params=pltpu.CompilerParams(use_tc_tiling_on_sc=False),
    scratch_types=[
        pltpu.VMEM((GPT, G), jnp.int32),
        pltpu.VMEM((GPT, G), jnp.int32),
        pltpu.VMEM((U, G, W8), jnp.float32),
        pltpu.VMEM_SHARED((NP, W8), jnp.float32),
        [pltpu.SemaphoreType.DMA] * U,
        pltpu.SemaphoreType.DMA,
    ],
)
def _sc_pass2(r_hbm, src_hbm, dst_hbm, z8_hbm,
              segp_hbm,
              src_v, dst_v, rows_v, seg_s, gsems, ssem):
    cid = lax.axis_index("c")
    sid = lax.axis_index("s")
    wid = sid * NC + cid

    rb = sid * ZMAIN
    pltpu.sync_copy(z8_hbm, seg_s.at[pl.ds(rb, ZMAIN)])

    @pl.when(sid == NS - 1)
    def _():
        tb = NS * ZMAIN
        pltpu.sync_copy(z8_hbm.at[pl.ds(0, NP - tb)], seg_s.at[pl.ds(tb, NP - tb)])

    pltpu.sync_copy(src_hbm.at[pl.ds(wid * GPT, GPT)], src_v)
    pltpu.sync_copy(dst_hbm.at[pl.ds(wid * GPT, GPT)], dst_v)

    plsc.subcore_barrier()

    NB = GPT // U

    def body(i, carry):
        b = i * U
        dg = [
            pltpu.async_copy(
                r_hbm.at[src_v.at[b + k]], rows_v.at[k], gsems[k])
            for k in range(U)
        ]
        ds = []
        for k in range(U):
            dg[k].wait()
            ds.append(pltpu.async_copy(
                rows_v.at[k], seg_s.at[dst_v.at[b + k]], ssem, add=True))
        for d in ds:
            d.wait()
        return carry

    lax.fori_loop(0, NB, body, 0)
    plsc.subcore_barrier()

    pltpu.sync_copy(seg_s.at[pl.ds(rb, ZMAIN)], segp_hbm.at[pl.ds(cid * N + rb, ZMAIN)])

    @pl.when(sid == NS - 1)
    def _():
        tb = NS * ZMAIN
        pltpu.sync_copy(seg_s.at[pl.ds(tb, N - tb)], segp_hbm.at[pl.ds(cid * N + tb, N - tb)])


BLK = 1000


def _tc_dense_body(agg_ref, cnt_ref, x_ref, w1l_ref, w1r_ref, b1_ref,
                   w2l_ref, w2r_ref, b2_ref, wlin_ref, blin_ref,
                   r8_ref, s_ref):
    a = agg_ref[...]
    c = cnt_ref[...]
    cnt = c[0, :, 0] + c[1, :, 0]
    inv = 1.0 / jnp.maximum(cnt, 1.0)
    aggm = jnp.concatenate([a[0], a[1]], axis=1) * inv[:, None]
    f32 = jnp.float32
    dn = (((1,), (1,)), ((), ()))
    h = lax.dot_general(aggm, w1l_ref[...], dn, preferred_element_type=f32)
    h = h + lax.dot_general(x_ref[...], w1r_ref[...], dn, preferred_element_type=f32)
    h = jnp.maximum(h + b1_ref[...][None, :], 0.0)
    wlin = wlin_ref[...]
    dn2 = (((1,), (0,)), ((), ()))
    wl2 = lax.dot_general(wlin, w2l_ref[...], dn2, preferred_element_type=f32)
    wr2 = lax.dot_general(wlin, w2r_ref[...], dn2, preferred_element_type=f32)
    c2 = jnp.sum(wlin * b2_ref[...][None, :], axis=1) + blin_ref[...]
    r = lax.dot_general(h, wl2, dn, preferred_element_type=f32)
    r8_ref[...] = jnp.concatenate([r, jnp.zeros((BLK, W8 - 2), f32)], axis=1)
    s_ref[...] = lax.dot_general(h, wr2, dn, preferred_element_type=f32) + c2[None, :]


def _tc_final_body(seg_ref, cnt_ref, s_ref, out_ref):
    sp = seg_ref[...]
    c = cnt_ref[...]
    cnt = c[0, :, 0] + c[1, :, 0]
    inv = 1.0 / jnp.maximum(cnt, 1.0)
    seg = sp[0, :, 0:2] + sp[1, :, 0:2]
    out_ref[...] = seg * inv[:, None] + s_ref[...]


def kernel(x, edge_index, W1_l, W1_r, b1, W2_l, W2_r, b2, W_lin, b_lin):
    ei = edge_index.astype(jnp.int32)
    npad = EPAD - E
    src = jnp.concatenate([ei[0], jnp.zeros((npad,), jnp.int32)]).reshape(NGROUPS, G)
    dst = jnp.concatenate([ei[1], jnp.full((npad,), N, jnp.int32)]).reshape(NGROUPS, G)
    srcx = jnp.concatenate([src, src + N], axis=0)
    xs = jnp.concatenate([x[:, :DH], x[:, DH:]], axis=0)
    z64 = jnp.zeros((ZMAIN, DH), jnp.float32)
    z8 = jnp.zeros((ZMAIN, W8), jnp.float32)
    ones8 = jnp.ones((G, W8), jnp.float32)

    aggp, cntp = _sc_pass1(xs, srcx, dst, z64, z8, ones8)
    aggp = aggp.reshape(NC, N, DH)
    cntp = cntp.reshape(NC, N, W8)

    grid = N // BLK
    full = lambda shape: pl.BlockSpec(shape, lambda i: tuple(0 for _ in shape))
    r8, s = pl.pallas_call(
        _tc_dense_body,
        grid=(grid,),
        in_specs=[
            pl.BlockSpec((NC, BLK, DH), lambda i: (0, i, 0)),
            pl.BlockSpec((NC, BLK, W8), lambda i: (0, i, 0)),
            pl.BlockSpec((BLK, D), lambda i: (i, 0)),
            full((D, D)), full((D, D)), full((D,)),
            full((4, D)), full((4, D)), full((4,)),
            full((2, 4)), full((2,)),
        ],
        out_specs=[
            pl.BlockSpec((BLK, W8), lambda i: (i, 0)),
            pl.BlockSpec((BLK, 2), lambda i: (i, 0)),
        ],
        out_shape=[
            jax.ShapeDtypeStruct((N, W8), jnp.float32),
            jax.ShapeDtypeStruct((N, 2), jnp.float32),
        ],
    )(aggp, cntp, x, W1_l, W1_r, b1, W2_l, W2_r, b2, W_lin, b_lin)

    (segp,) = _sc_pass2(r8, src, dst, z8)
    segp = segp.reshape(NC, N, W8)

    out = pl.pallas_call(
        _tc_final_body,
        grid=(grid,),
        in_specs=[
            pl.BlockSpec((NC, BLK, W8), lambda i: (0, i, 0)),
            pl.BlockSpec((NC, BLK, W8), lambda i: (0, i, 0)),
            pl.BlockSpec((BLK, 2), lambda i: (i, 0)),
        ],
        out_specs=pl.BlockSpec((BLK, 2), lambda i: (i, 0)),
        out_shape=jax.ShapeDtypeStruct((N, 2), jnp.float32),
    )(segp, cntp, s)
    return out


# speedup vs baseline: 1.0026x; 1.0026x over previous
import functools

import jax
import jax.numpy as jnp
from jax import lax
from jax.experimental import pallas as pl
from jax.experimental.pallas import tpu as pltpu
from jax.experimental.pallas import tpu_sc as plsc

N = 10000
E = 320000
D = 128
DH = D // 2
NC = 2
NS = 16
NW = NC * NS
G = 128
NGROUPS = 2560
EPAD = NGROUPS * G
GPC = NGROUPS // NS
GPT = NGROUPS // NW
NP = N + 8
ZMAIN = 624
W8 = 8
U = 5
U2 = 10

_mesh = plsc.VectorSubcoreMesh(core_axis_name="c", subcore_axis_name="s")


@functools.partial(
    pl.kernel,
    out_type=[
        jax.ShapeDtypeStruct((NC * N, DH), jnp.float32),
        jax.ShapeDtypeStruct((NC * N, W8), jnp.float32),
    ],
    mesh=_mesh,
    compiler_params=pltpu.CompilerParams(use_tc_tiling_on_sc=False),
    scratch_types=[
        pltpu.VMEM((GPC, G), jnp.int32),
        pltpu.VMEM((GPC, G), jnp.int32),
        pltpu.VMEM((U, G, DH), jnp.float32),
        pltpu.VMEM((G, W8), jnp.float32),
        pltpu.VMEM_SHARED((NP, DH), jnp.float32),
        pltpu.VMEM_SHARED((NP, W8), jnp.float32),
        [pltpu.SemaphoreType.DMA] * U,
        pltpu.SemaphoreType.DMA,
        pltpu.SemaphoreType.DMA,
    ],
)
def _sc_pass1(xs_hbm, srcx_hbm, dst_hbm, z64_hbm, z8_hbm, ones_hbm,
              aggp_hbm, cntp_hbm,
              src_v, dst_v, rows_v, ones_v, acc_s, cnt_s, gsems, ssem, osem):
    cid = lax.axis_index("c")
    sid = lax.axis_index("s")

    rb = sid * ZMAIN
    pltpu.sync_copy(z64_hbm, acc_s.at[pl.ds(rb, ZMAIN)])
    pltpu.sync_copy(z8_hbm, cnt_s.at[pl.ds(rb, ZMAIN)])

    @pl.when(sid == NS - 1)
    def _():
        tb = NS * ZMAIN
        pltpu.sync_copy(z64_hbm.at[pl.ds(0, NP - tb)], acc_s.at[pl.ds(tb, NP - tb)])
        pltpu.sync_copy(z8_hbm.at[pl.ds(0, NP - tb)], cnt_s.at[pl.ds(tb, NP - tb)])

    gb = sid * GPC
    pltpu.sync_copy(srcx_hbm.at[pl.ds(cid * NGROUPS + gb, GPC)], src_v)
    pltpu.sync_copy(dst_hbm.at[pl.ds(gb, GPC)], dst_v)
    pltpu.sync_copy(ones_hbm, ones_v)

    plsc.subcore_barrier()

    NB = GPC // U

    def body(i, carry):
        b = i * U
        dg = [
            pltpu.async_copy(
                xs_hbm.at[src_v.at[b + k]], rows_v.at[k], gsems[k])
            for k in range(U)
        ]
        ds = []
        for k in range(U):
            dg[k].wait()
            ds.append(pltpu.async_copy(
                rows_v.at[k], acc_s.at[dst_v.at[b + k]], ssem, add=True))

        @pl.when((cid == 0) == (i < NB // 2))
        def _():
            do = [
                pltpu.async_copy(
                    ones_v, cnt_s.at[dst_v.at[b + k]], osem, add=True)
                for k in range(U)
            ]
            for d in do:
                d.wait()

        for d in ds:
            d.wait()
        return carry

    lax.fori_loop(0, NB, body, 0)
    plsc.subcore_barrier()

    pltpu.sync_copy(acc_s.at[pl.ds(rb, ZMAIN)], aggp_hbm.at[pl.ds(cid * N + rb, ZMAIN)])
    pltpu.sync_copy(cnt_s.at[pl.ds(rb, ZMAIN)], cntp_hbm.at[pl.ds(cid * N + rb, ZMAIN)])

    @pl.when(sid == NS - 1)
    def _():
        tb = NS * ZMAIN
        pltpu.sync_copy(acc_s.at[pl.ds(tb, N - tb)], aggp_hbm.at[pl.ds(cid * N + tb, N - tb)])
        pltpu.sync_copy(cnt_s.at[pl.ds(tb, N - tb)], cntp_hbm.at[pl.ds(cid * N + tb, N - tb)])


@functools.partial(
    pl.kernel,
    out_type=[jax.ShapeDtypeStruct((NC * N, W8), jnp.float32)],
    mesh=_mesh,
    compiler_params=pltpu.CompilerParams(use_tc_tiling_on_sc=False),
    scratch_types=[
        pltpu.VMEM((GPT, G), jnp.int32),
        pltpu.VMEM((GPT, G), jnp.int32),
        pltpu.VMEM((U2, G, W8), jnp.float32),
        pltpu.VMEM_SHARED((NP, W8), jnp.float32),
        [pltpu.SemaphoreType.DMA] * U2,
        pltpu.SemaphoreType.DMA,
    ],
)
def _sc_pass2(r_hbm, src_hbm, dst_hbm, z8_hbm,
              segp_hbm,
              src_v, dst_v, rows_v, seg_s, gsems, ssem):
    cid = lax.axis_index("c")
    sid = lax.axis_index("s")
    wid = sid * NC + cid

    rb = sid * ZMAIN
    pltpu.sync_copy(z8_hbm, seg_s.at[pl.ds(rb, ZMAIN)])

    @pl.when(sid == NS - 1)
    def _():
        tb = NS * ZMAIN
        pltpu.sync_copy(z8_hbm.at[pl.ds(0, NP - tb)], seg_s.at[pl.ds(tb, NP - tb)])

    pltpu.sync_copy(src_hbm.at[pl.ds(wid * GPT, GPT)], src_v)
    pltpu.sync_copy(dst_hbm.at[pl.ds(wid * GPT, GPT)], dst_v)

    plsc.subcore_barrier()

    NB = GPT // U2

    def body(i, carry):
        b = i * U2
        dg = [
            pltpu.async_copy(
                r_hbm.at[src_v.at[b + k]], rows_v.at[k], gsems[k])
            for k in range(U2)
        ]
        ds = []
        for k in range(U2):
            dg[k].wait()
            ds.append(pltpu.async_copy(
                rows_v.at[k], seg_s.at[dst_v.at[b + k]], ssem, add=True))
        for d in ds:
            d.wait()
        return carry

    lax.fori_loop(0, NB, body, 0)
    plsc.subcore_barrier()

    pltpu.sync_copy(seg_s.at[pl.ds(rb, ZMAIN)], segp_hbm.at[pl.ds(cid * N + rb, ZMAIN)])

    @pl.when(sid == NS - 1)
    def _():
        tb = NS * ZMAIN
        pltpu.sync_copy(seg_s.at[pl.ds(tb, N - tb)], segp_hbm.at[pl.ds(cid * N + tb, N - tb)])


BLK = 1000


def _tc_dense_body(agg_ref, cnt_ref, x_ref, w1l_ref, w1r_ref, b1_ref,
                   w2l_ref, w2r_ref, b2_ref, wlin_ref, blin_ref,
                   r8_ref, s_ref):
    a = agg_ref[...]
    c = cnt_ref[...]
    cnt = c[0, :, 0] + c[1, :, 0]
    inv = 1.0 / jnp.maximum(cnt, 1.0)
    aggm = jnp.concatenate([a[0], a[1]], axis=1) * inv[:, None]
    f32 = jnp.float32
    dn = (((1,), (1,)), ((), ()))
    h = lax.dot_general(aggm, w1l_ref[...], dn, preferred_element_type=f32)
    h = h + lax.dot_general(x_ref[...], w1r_ref[...], dn, preferred_element_type=f32)
    h = jnp.maximum(h + b1_ref[...][None, :], 0.0)
    wlin = wlin_ref[...]
    dn2 = (((1,), (0,)), ((), ()))
    wl2 = lax.dot_general(wlin, w2l_ref[...], dn2, preferred_element_type=f32)
    wr2 = lax.dot_general(wlin, w2r_ref[...], dn2, preferred_element_type=f32)
    c2 = jnp.sum(wlin * b2_ref[...][None, :], axis=1) + blin_ref[...]
    r = lax.dot_general(h, wl2, dn, preferred_element_type=f32)
    r8_ref[...] = jnp.concatenate([r, jnp.zeros((BLK, W8 - 2), f32)], axis=1)
    s_ref[...] = lax.dot_general(h, wr2, dn, preferred_element_type=f32) + c2[None, :]


def _tc_final_body(seg_ref, cnt_ref, s_ref, out_ref):
    sp = seg_ref[...]
    c = cnt_ref[...]
    cnt = c[0, :, 0] + c[1, :, 0]
    inv = 1.0 / jnp.maximum(cnt, 1.0)
    seg = sp[0, :, 0:2] + sp[1, :, 0:2]
    out_ref[...] = seg * inv[:, None] + s_ref[...]


def kernel(x, edge_index, W1_l, W1_r, b1, W2_l, W2_r, b2, W_lin, b_lin):
    ei = edge_index.astype(jnp.int32)
    npad = EPAD - E
    src = jnp.concatenate([ei[0], jnp.zeros((npad,), jnp.int32)]).reshape(NGROUPS, G)
    dst = jnp.concatenate([ei[1], jnp.full((npad,), N, jnp.int32)]).reshape(NGROUPS, G)
    srcx = jnp.concatenate([src, src + N], axis=0)
    xs = jnp.concatenate([x[:, :DH], x[:, DH:]], axis=0)
    z64 = jnp.zeros((ZMAIN, DH), jnp.float32)
    z8 = jnp.zeros((ZMAIN, W8), jnp.float32)
    ones8 = jnp.ones((G, W8), jnp.float32)

    aggp, cntp = _sc_pass1(xs, srcx, dst, z64, z8, ones8)
    aggp = aggp.reshape(NC, N, DH)
    cntp = cntp.reshape(NC, N, W8)

    grid = N // BLK
    full = lambda shape: pl.BlockSpec(shape, lambda i: tuple(0 for _ in shape))
    r8, s = pl.pallas_call(
        _tc_dense_body,
        grid=(grid,),
        in_specs=[
            pl.BlockSpec((NC, BLK, DH), lambda i: (0, i, 0)),
            pl.BlockSpec((NC, BLK, W8), lambda i: (0, i, 0)),
            pl.BlockSpec((BLK, D), lambda i: (i, 0)),
            full((D, D)), full((D, D)), full((D,)),
            full((4, D)), full((4, D)), full((4,)),
            full((2, 4)), full((2,)),
        ],
        out_specs=[
            pl.BlockSpec((BLK, W8), lambda i: (i, 0)),
            pl.BlockSpec((BLK, 2), lambda i: (i, 0)),
        ],
        out_shape=[
            jax.ShapeDtypeStruct((N, W8), jnp.float32),
            jax.ShapeDtypeStruct((N, 2), jnp.float32),
        ],
    )(aggp, cntp, x, W1_l, W1_r, b1, W2_l, W2_r, b2, W_lin, b_lin)

    (segp,) = _sc_pass2(r8, src, dst, z8)
    segp = segp.reshape(NC, N, W8)

    out = pl.pallas_call(
        _tc_final_body,
        grid=(grid,),
        in_specs=[
            pl.BlockSpec((NC, BLK, W8), lambda i: (0, i, 0)),
            pl.BlockSpec((NC, BLK, W8), lambda i: (0, i, 0)),
            pl.BlockSpec((BLK, 2), lambda i: (i, 0)),
        ],
        out_specs=pl.BlockSpec((BLK, 2), lambda i: (i, 0)),
        out_shape=jax.ShapeDtypeStruct((N, 2), jnp.float32),
    )(segp, cntp, s)
    return out
